# baseline (speedup 1.0000x reference)
import jax
import jax.numpy as jnp
from jax.experimental import pallas as pl
from jax.experimental.pallas import tpu as pltpu

_DELTA = 0.1
_EPS = 1e-12
_BR = 256
_BC = 2048


def _ghm_body(x_rows_ref, y_rows_ref, x_cols_ref, y_cols_ref, pw_ref,
              wsum_ref, psum_ref, gcols_ref):
    i = pl.program_id(0)
    n = x_cols_ref.shape[1]

    @pl.when(i == 0)
    def _init():
        xc = x_cols_ref[...]
        yc = y_cols_ref[...]
        gcols_ref[...] = jnp.abs(jax.nn.sigmoid(xc) - yc)
        wsum_ref[0, 0] = 0.0
        psum_ref[0, 0] = 0.0

    x_r = x_rows_ref[0]
    y_r = y_rows_ref[0]
    g_r = jnp.abs(jax.nn.sigmoid(x_r) - y_r)

    def col_step(c, acc):
        off = pl.multiple_of(c * _BC, _BC)
        gc = gcols_ref[:, pl.ds(off, _BC)]
        m = (jnp.abs(g_r - gc) <= _DELTA).astype(jnp.float32)
        return acc + jnp.sum(m, axis=1, keepdims=True)

    cnt = jax.lax.fori_loop(0, n // _BC, col_step,
                            jnp.zeros((_BR, 1), jnp.float32))

    gd = cnt / _DELTA
    beta = n / (gd + _EPS)

    pw = pw_ref[0, 0]
    pe = pw * y_r * jax.nn.softplus(-x_r) + (1.0 - y_r) * jax.nn.softplus(x_r)

    wsum_ref[0, 0] += jnp.sum(beta * pe)
    psum_ref[0, 0] += jnp.sum(pe)


def kernel(logits, targets, pos_weight):
    x = logits.reshape(-1).astype(jnp.float32)
    y = targets.reshape(-1).astype(jnp.float32)
    n = x.shape[0]
    g = n // _BR

    x_rows = x.reshape(g, _BR, 1)
    y_rows = y.reshape(g, _BR, 1)
    x_cols = x.reshape(1, n)
    y_cols = y.reshape(1, n)
    pw = jnp.asarray(pos_weight, jnp.float32).reshape(1, 1)

    wsum, psum = pl.pallas_call(
        _ghm_body,
        grid=(g,),
        in_specs=[
            pl.BlockSpec((1, _BR, 1), lambda i: (i, 0, 0)),
            pl.BlockSpec((1, _BR, 1), lambda i: (i, 0, 0)),
            pl.BlockSpec((1, n), lambda i: (0, 0)),
            pl.BlockSpec((1, n), lambda i: (0, 0)),
            pl.BlockSpec(memory_space=pltpu.SMEM),
        ],
        out_specs=[
            pl.BlockSpec((1, 1), lambda i: (0, 0), memory_space=pltpu.SMEM),
            pl.BlockSpec((1, 1), lambda i: (0, 0), memory_space=pltpu.SMEM),
        ],
        out_shape=[
            jax.ShapeDtypeStruct((1, 1), jnp.float32),
            jax.ShapeDtypeStruct((1, 1), jnp.float32),
        ],
        scratch_shapes=[pltpu.VMEM((1, n), jnp.float32)],
        compiler_params=pltpu.CompilerParams(
            dimension_semantics=("arbitrary",),
        ),
        name="ghm_bce",
    )(x_rows, y_rows, x_cols, y_cols, pw)

    inv_n = jnp.float32(1.0 / n)
    return wsum[0, 0] * inv_n, psum[0, 0] * inv_n


# speedup vs baseline: 1.1513x; 1.1513x over previous
import jax
import jax.numpy as jnp
from jax.experimental import pallas as pl
from jax.experimental.pallas import tpu as pltpu

_DELTA = 0.1
_EPS = 1e-12
_BR = 256
_BC = 2048


def _ghm_body(x_rows_ref, y_rows_ref, x_cols_ref, y_cols_ref, pw_ref,
              wsum_ref, psum_ref, gcols_ref):
    i = pl.program_id(0)
    n = x_cols_ref.shape[1]

    @pl.when(i == 0)
    def _init():
        xc = x_cols_ref[...]
        yc = y_cols_ref[...]
        gcols_ref[...] = jnp.abs(jax.nn.sigmoid(xc) - yc)
        wsum_ref[0, 0] = 0.0
        psum_ref[0, 0] = 0.0

    x_r = x_rows_ref[0]
    y_r = y_rows_ref[0]
    g_r = jnp.abs(jax.nn.sigmoid(x_r) - y_r)

    def col_step(c, acc):
        off = pl.multiple_of(c * _BC, _BC)
        gc = gcols_ref[:, pl.ds(off, _BC)]
        m = (jnp.abs(g_r - gc) <= _DELTA).astype(jnp.float32)
        t = m[:, 0:128]
        for s in range(1, _BC // 128):
            t = t + m[:, s * 128:(s + 1) * 128]
        return acc + t

    cnt128 = jax.lax.fori_loop(0, n // _BC, col_step,
                               jnp.zeros((_BR, 128), jnp.float32))
    cnt = jnp.sum(cnt128, axis=1, keepdims=True)

    gd = cnt / _DELTA
    beta = n / (gd + _EPS)

    pw = pw_ref[0, 0]
    pe = pw * y_r * jax.nn.softplus(-x_r) + (1.0 - y_r) * jax.nn.softplus(x_r)

    wsum_ref[0, 0] += jnp.sum(beta * pe)
    psum_ref[0, 0] += jnp.sum(pe)


def kernel(logits, targets, pos_weight):
    x = logits.reshape(-1).astype(jnp.float32)
    y = targets.reshape(-1).astype(jnp.float32)
    n = x.shape[0]
    g = n // _BR

    x_rows = x.reshape(g, _BR, 1)
    y_rows = y.reshape(g, _BR, 1)
    x_cols = x.reshape(1, n)
    y_cols = y.reshape(1, n)
    pw = jnp.asarray(pos_weight, jnp.float32).reshape(1, 1)

    wsum, psum = pl.pallas_call(
        _ghm_body,
        grid=(g,),
        in_specs=[
            pl.BlockSpec((1, _BR, 1), lambda i: (i, 0, 0)),
            pl.BlockSpec((1, _BR, 1), lambda i: (i, 0, 0)),
            pl.BlockSpec((1, n), lambda i: (0, 0)),
            pl.BlockSpec((1, n), lambda i: (0, 0)),
            pl.BlockSpec(memory_space=pltpu.SMEM),
        ],
        out_specs=[
            pl.BlockSpec((1, 1), lambda i: (0, 0), memory_space=pltpu.SMEM),
            pl.BlockSpec((1, 1), lambda i: (0, 0), memory_space=pltpu.SMEM),
        ],
        out_shape=[
            jax.ShapeDtypeStruct((1, 1), jnp.float32),
            jax.ShapeDtypeStruct((1, 1), jnp.float32),
        ],
        scratch_shapes=[pltpu.VMEM((1, n), jnp.float32)],
        compiler_params=pltpu.CompilerParams(
            dimension_semantics=("arbitrary",),
        ),
        name="ghm_bce",
    )(x_rows, y_rows, x_cols, y_cols, pw)

    inv_n = jnp.float32(1.0 / n)
    return wsum[0, 0] * inv_n, psum[0, 0] * inv_n
